# baseline (speedup 1.0000x reference)
import jax
import jax.numpy as jnp
from jax.experimental import pallas as pl
from jax.experimental.pallas import tpu as pltpu


def _fused_mlp_kernel(x_ref, w1_ref, b1_ref, w2_ref, b2_ref, w3_ref, b3_ref,
                      o_ref):
    dot = lambda w, h: jax.lax.dot_general(
        w, h, (((1,), (0,)), ((), ())), preferred_element_type=jnp.float32)
    h = jnp.maximum(dot(w1_ref[...], x_ref[0]) + b1_ref[...], 0.0)
    h = jnp.maximum(dot(w2_ref[...], h) + b2_ref[...], 0.0)
    o_ref[0] = jnp.maximum(
        dot(w3_ref[...], h.astype(jnp.bfloat16)) + b3_ref[...], 0.0)


def kernel(x, scale0_params, scale1_params, scale2_params, fusion_params):
    del scale0_params, scale1_params, scale2_params
    (w1, b1), (w2, b2), (w3, b3) = fusion_params
    B, C_in, N = x.shape
    C_out = w3.shape[0]

    full = lambda shape: pl.BlockSpec(shape, lambda b: (0,) * len(shape))
    return pl.pallas_call(
        _fused_mlp_kernel,
        grid=(B,),
        in_specs=[
            pl.BlockSpec((1, C_in, N), lambda b: (b, 0, 0)),
            full(w1.shape), full((w1.shape[0], 1)),
            full(w2.shape), full((w2.shape[0], 1)),
            full(w3.shape), full((w3.shape[0], 1)),
        ],
        out_specs=pl.BlockSpec((1, C_out, N), lambda b: (b, 0, 0)),
        out_shape=jax.ShapeDtypeStruct((B, C_out, N), jnp.float32),
        compiler_params=pltpu.CompilerParams(
            dimension_semantics=("parallel",)),
    )(x, w1, b1[:, None], w2, b2[:, None],
      w3.astype(jnp.bfloat16), b3[:, None])


# speedup vs baseline: 1.0484x; 1.0484x over previous
import jax
import jax.numpy as jnp
from jax.experimental import pallas as pl
from jax.experimental.pallas import tpu as pltpu


def _make_body(num_b):
    def body(x_ref, w1_ref, b1_ref, w2_ref, b2_ref, w3_ref, b3_ref,
             o_ref, s_ref, sem):
        b = pl.program_id(0)
        slot = jax.lax.rem(b, 2)

        @pl.when(b >= 2)
        def _():
            pltpu.make_async_copy(
                s_ref.at[slot], o_ref.at[b - 2], sem.at[slot]).wait()

        dot = lambda w, h: jax.lax.dot_general(
            w, h, (((1,), (0,)), ((), ())),
            preferred_element_type=jnp.float32)
        h = jnp.maximum(dot(w1_ref[...], x_ref[0]) + b1_ref[...], 0.0)
        h = jnp.maximum(dot(w2_ref[...], h) + b2_ref[...], 0.0)
        s_ref[slot] = jnp.maximum(dot(w3_ref[...], h) + b3_ref[...], 0.0)

        pltpu.make_async_copy(s_ref.at[slot], o_ref.at[b], sem.at[slot]).start()

        @pl.when(b == num_b - 1)
        def _():
            pltpu.make_async_copy(
                s_ref.at[1 - slot], o_ref.at[b - 1], sem.at[1 - slot]).wait()
            pltpu.make_async_copy(
                s_ref.at[slot], o_ref.at[b], sem.at[slot]).wait()

    return body


def kernel(x, scale0_params, scale1_params, scale2_params, fusion_params):
    del scale0_params, scale1_params, scale2_params
    (w1, b1), (w2, b2), (w3, b3) = fusion_params
    B, C_in, N = x.shape
    C_out = w3.shape[0]

    full = lambda shape: pl.BlockSpec(shape, lambda b: (0,) * len(shape))
    return pl.pallas_call(
        _make_body(B),
        grid=(B,),
        in_specs=[
            pl.BlockSpec((1, C_in, N), lambda b: (b, 0, 0)),
            full(w1.shape), full((w1.shape[0], 1)),
            full(w2.shape), full((w2.shape[0], 1)),
            full(w3.shape), full((w3.shape[0], 1)),
        ],
        out_specs=pl.BlockSpec(memory_space=pltpu.MemorySpace.HBM),
        out_shape=jax.ShapeDtypeStruct((B, C_out, N), jnp.float32),
        scratch_shapes=[
            pltpu.VMEM((2, C_out, N), jnp.float32),
            pltpu.SemaphoreType.DMA((2,)),
        ],
        compiler_params=pltpu.CompilerParams(
            dimension_semantics=("arbitrary",)),
    )(x, w1, b1[:, None], w2, b2[:, None], w3, b3[:, None])


# speedup vs baseline: 1.0601x; 1.0112x over previous
import jax
import jax.numpy as jnp
from jax.experimental import pallas as pl
from jax.experimental.pallas import tpu as pltpu

_NCHUNK = 4


def _make_body(num_b, nchunk, tile_c):
    def body(x_ref, w1_ref, b1_ref, w2_ref, b2_ref, w3_ref, b3_ref,
             o_ref, s_ref, sem):
        b = pl.program_id(0)
        dot = lambda w, h: jax.lax.dot_general(
            w, h, (((1,), (0,)), ((), ())),
            preferred_element_type=jnp.float32)
        h = jnp.maximum(dot(w1_ref[...], x_ref[0]) + b1_ref[...], 0.0)
        h = jnp.maximum(dot(w2_ref[...], h) + b2_ref[...], 0.0)

        for j in range(nchunk):
            cs = pl.ds(j * tile_c, tile_c)

            @pl.when(b >= 1)
            def _():
                pltpu.make_async_copy(
                    s_ref.at[j], o_ref.at[b - 1, cs, :], sem.at[j]).wait()

            s_ref[j] = jnp.maximum(
                dot(w3_ref[cs, :], h) + b3_ref[cs, :], 0.0)
            pltpu.make_async_copy(
                s_ref.at[j], o_ref.at[b, cs, :], sem.at[j]).start()

        @pl.when(b == num_b - 1)
        def _():
            for j in range(nchunk):
                cs = pl.ds(j * tile_c, tile_c)
                pltpu.make_async_copy(
                    s_ref.at[j], o_ref.at[b, cs, :], sem.at[j]).wait()

    return body


def kernel(x, scale0_params, scale1_params, scale2_params, fusion_params):
    del scale0_params, scale1_params, scale2_params
    (w1, b1), (w2, b2), (w3, b3) = fusion_params
    B, C_in, N = x.shape
    C_out = w3.shape[0]
    tile_c = C_out // _NCHUNK

    full = lambda shape: pl.BlockSpec(shape, lambda b: (0,) * len(shape))
    return pl.pallas_call(
        _make_body(B, _NCHUNK, tile_c),
        grid=(B,),
        in_specs=[
            pl.BlockSpec((1, C_in, N), lambda b: (b, 0, 0)),
            full(w1.shape), full((w1.shape[0], 1)),
            full(w2.shape), full((w2.shape[0], 1)),
            full(w3.shape), full((w3.shape[0], 1)),
        ],
        out_specs=pl.BlockSpec(memory_space=pltpu.MemorySpace.HBM),
        out_shape=jax.ShapeDtypeStruct((B, C_out, N), jnp.float32),
        scratch_shapes=[
            pltpu.VMEM((_NCHUNK, tile_c, N), jnp.float32),
            pltpu.SemaphoreType.DMA((_NCHUNK,)),
        ],
        compiler_params=pltpu.CompilerParams(
            dimension_semantics=("arbitrary",)),
    )(x, w1, b1[:, None], w2, b2[:, None], w3, b3[:, None])


# speedup vs baseline: 1.2197x; 1.1505x over previous
import jax
import jax.numpy as jnp
from jax.experimental import pallas as pl
from jax.experimental.pallas import tpu as pltpu

_NCHUNK = 4


def _make_body(num_b, nchunk, tile_c):
    def body(x_ref, w1_ref, b1_ref, w2_ref, b2_ref, w3_ref, b3_ref,
             o_ref, s_ref, sem):
        b = pl.program_id(0)
        dot = lambda w, h: jax.lax.dot_general(
            w, h, (((1,), (0,)), ((), ())),
            preferred_element_type=jnp.float32)
        h = jnp.maximum(
            dot(w1_ref[...], x_ref[0]) + b1_ref[...][:, None], 0.0)
        h = jnp.maximum(dot(w2_ref[...], h) + b2_ref[...][:, None], 0.0)

        for j in range(nchunk):
            cs = pl.ds(j * tile_c, tile_c)

            @pl.when(b >= 1)
            def _():
                pltpu.make_async_copy(
                    s_ref.at[j], o_ref.at[b - 1, cs, :], sem.at[j]).wait()

            s_ref[j] = jnp.maximum(
                dot(w3_ref[cs, :], h) + b3_ref[cs][:, None], 0.0)
            pltpu.make_async_copy(
                s_ref.at[j], o_ref.at[b, cs, :], sem.at[j]).start()

        @pl.when(b == num_b - 1)
        def _():
            for j in range(nchunk):
                cs = pl.ds(j * tile_c, tile_c)
                pltpu.make_async_copy(
                    s_ref.at[j], o_ref.at[b, cs, :], sem.at[j]).wait()

    return body


def kernel(x, scale0_params, scale1_params, scale2_params, fusion_params):
    del scale0_params, scale1_params, scale2_params
    (w1, b1), (w2, b2), (w3, b3) = fusion_params
    B, C_in, N = x.shape
    C_out = w3.shape[0]
    tile_c = C_out // _NCHUNK

    full = lambda shape: pl.BlockSpec(shape, lambda b: (0,) * len(shape))
    return pl.pallas_call(
        _make_body(B, _NCHUNK, tile_c),
        grid=(B,),
        in_specs=[
            pl.BlockSpec((1, C_in, N), lambda b: (b, 0, 0)),
            full(w1.shape), full(b1.shape),
            full(w2.shape), full(b2.shape),
            full(w3.shape), full(b3.shape),
        ],
        out_specs=pl.BlockSpec(memory_space=pltpu.MemorySpace.HBM),
        out_shape=jax.ShapeDtypeStruct((B, C_out, N), jnp.float32),
        scratch_shapes=[
            pltpu.VMEM((_NCHUNK, tile_c, N), jnp.float32),
            pltpu.SemaphoreType.DMA((_NCHUNK,)),
        ],
        compiler_params=pltpu.CompilerParams(
            dimension_semantics=("arbitrary",)),
    )(x, w1, b1, w2, b2, w3, b3)


# speedup vs baseline: 1.5036x; 1.2327x over previous
import jax
import jax.numpy as jnp
from jax.experimental import pallas as pl
from jax.experimental.pallas import tpu as pltpu

_NCHUNK = 4


def _make_body(num_b, nchunk, tile_c):
    def body(x_ref, w1_ref, b1_ref, w2_ref, b2_ref, w3_ref, b3_ref,
             o_ref, s_ref, sem):
        b = pl.program_id(0)
        dot_t = lambda wt, h: jax.lax.dot_general(
            wt, h, (((0,), (0,)), ((), ())),
            preferred_element_type=jnp.float32)
        dot = lambda w, h: jax.lax.dot_general(
            w, h, (((1,), (0,)), ((), ())),
            preferred_element_type=jnp.float32)
        h = jnp.maximum(
            dot_t(w1_ref[...], x_ref[:, b, :]) + b1_ref[...][:, None], 0.0)
        h = jnp.maximum(dot_t(w2_ref[...], h) + b2_ref[...][:, None], 0.0)

        for j in range(nchunk):
            cs = pl.ds(j * tile_c, tile_c)

            @pl.when(b >= 1)
            def _():
                pltpu.make_async_copy(
                    s_ref.at[j], o_ref.at[b - 1, cs, :], sem.at[j]).wait()

            s_ref[j] = jnp.maximum(
                dot(w3_ref[cs, :], h) + b3_ref[cs][:, None], 0.0)
            pltpu.make_async_copy(
                s_ref.at[j], o_ref.at[b, cs, :], sem.at[j]).start()

        @pl.when(b == num_b - 1)
        def _():
            for j in range(nchunk):
                cs = pl.ds(j * tile_c, tile_c)
                pltpu.make_async_copy(
                    s_ref.at[j], o_ref.at[b, cs, :], sem.at[j]).wait()

    return body


def kernel(x, scale0_params, scale1_params, scale2_params, fusion_params):
    del scale0_params, scale1_params, scale2_params
    (w1, b1), (w2, b2), (w3, b3) = fusion_params
    B, C_in, N = x.shape
    C_out = w3.shape[0]
    tile_c = C_out // _NCHUNK
    xt = jnp.transpose(x, (1, 0, 2))
    w1t, w2t = w1.T, w2.T

    full = lambda shape: pl.BlockSpec(shape, lambda b: (0,) * len(shape))
    return pl.pallas_call(
        _make_body(B, _NCHUNK, tile_c),
        grid=(B,),
        in_specs=[
            pl.BlockSpec((C_in, B, N), lambda b: (0, 0, 0)),
            full(w1t.shape), full(b1.shape),
            full(w2t.shape), full(b2.shape),
            full(w3.shape), full(b3.shape),
        ],
        out_specs=pl.BlockSpec(memory_space=pltpu.MemorySpace.HBM),
        out_shape=jax.ShapeDtypeStruct((B, C_out, N), jnp.float32),
        scratch_shapes=[
            pltpu.VMEM((_NCHUNK, tile_c, N), jnp.float32),
            pltpu.SemaphoreType.DMA((_NCHUNK,)),
        ],
        compiler_params=pltpu.CompilerParams(
            dimension_semantics=("arbitrary",)),
    )(xt, w1t, b1, w2t, b2, w3, b3)
